# Initial kernel scaffold; baseline (speedup 1.0000x reference)
import jax
import jax.numpy as jnp
from jax.experimental import pallas as pl


def kernel(x, adjs, Wl0, bl0, Wr0, Wl1, bl1, Wr1, Wl2, bl2, Wr2):
    raise NotImplementedError("write your pallas kernel here")


# speedup vs baseline: 4.4779x; 4.4779x over previous
import functools

import jax
import jax.numpy as jnp
from jax import lax
from jax.experimental import pallas as pl
from jax.experimental.pallas import tpu as pltpu
from jax.experimental.pallas import tpu_sc as plsc

NN = 10000
NP = 10240
DD = 256
DH = 128
EE = 160000
CHUNK = 128
NCHUNKS = EE // CHUNK
NTILES = 16
ROWS_PER_TILE = NP // NTILES
RBLK = 1024


def _sc_aggregate(xf, src, dst):
    mesh = plsc.VectorSubcoreMesh(core_axis_name="c", subcore_axis_name="s",
                                  num_cores=2, num_subcores=NTILES)

    @functools.partial(
        pl.kernel,
        mesh=mesh,
        out_type=[
            jax.ShapeDtypeStruct((2 * NP, DH), jnp.float32),
            jax.ShapeDtypeStruct((NTILES, NP), jnp.float32),
        ],
        scratch_types=[
            pltpu.VMEM((CHUNK,), jnp.int32),
            pltpu.VMEM((CHUNK,), jnp.int32),
            pltpu.VMEM((CHUNK, DH), jnp.float32),
            pltpu.VMEM((NP,), jnp.float32),
            pltpu.VMEM_SHARED((NP, DH), jnp.float32),
            pltpu.SemaphoreType.DMA,
        ],
        compiler_params=pltpu.CompilerParams(needs_layout_passes=False),
    )
    def k(xf_hbm, src_hbm, dst_hbm, agg_hbm, degp_hbm,
          src_v, dst_v, rows_v, deg_v, acc_sh, sem):
        c = lax.axis_index("c")
        s = lax.axis_index("s")
        zero16 = jnp.zeros((16,), jnp.float32)
        ones16 = jnp.ones((16,), jnp.float32)

        def _zrow(i, carry):
            def _zcol(j, carry2):
                rows_v[i, pl.ds(j * 16, 16)] = zero16
                return carry2
            return lax.fori_loop(0, DH // 16, _zcol, carry)
        lax.fori_loop(0, CHUNK, _zrow, 0)

        def _zdeg(i, carry):
            deg_v[pl.ds(i * 16, 16)] = zero16
            return carry
        lax.fori_loop(0, NP // 16, _zdeg, 0)

        base = s * ROWS_PER_TILE
        for b in range(ROWS_PER_TILE // CHUNK):
            pltpu.sync_copy(rows_v, acc_sh.at[pl.ds(base + b * CHUNK, CHUNK)])
        plsc.subcore_barrier()

        c_off = c * NP

        def _edge_chunk(t, carry):
            chunk = s + t * NTILES

            @pl.when(chunk < NCHUNKS)
            def _():
                off = chunk * CHUNK
                pltpu.sync_copy(src_hbm.at[pl.ds(off, CHUNK)], src_v)
                pltpu.sync_copy(dst_hbm.at[pl.ds(off, CHUNK)], dst_v)
                for j in range(CHUNK // 16):
                    sl = pl.ds(j * 16, 16)
                    src_v[sl] = src_v[sl] + c_off
                pltpu.async_copy(xf_hbm.at[src_v], rows_v, sem).wait()
                pltpu.sync_copy(rows_v, acc_sh.at[dst_v], add=True)

                @pl.when(c == 0)
                def _():
                    for j in range(CHUNK // 16):
                        d16 = dst_v[pl.ds(j * 16, 16)]
                        plsc.addupdate_scatter(deg_v, [d16], ones16)
            return carry

        lax.fori_loop(0, (NCHUNKS + NTILES - 1) // NTILES, _edge_chunk, 0)
        plsc.subcore_barrier()

        pltpu.sync_copy(acc_sh.at[pl.ds(base, ROWS_PER_TILE)],
                        agg_hbm.at[pl.ds(c_off + base, ROWS_PER_TILE)])

        @pl.when(c == 0)
        def _():
            pltpu.sync_copy(deg_v, degp_hbm.at[s])

    return k(xf, src, dst)


def _tc_update(aggf, degp, xf, wl2, bl2d, wr2, last):
    nblk = NP // RBLK

    def body(dp_ref, a0_ref, a1_ref, x0_ref, x1_ref, wl_ref, wr_ref, b_ref,
             o_ref):
        deg = jnp.sum(dp_ref[...], axis=0)
        inv = 1.0 / jnp.maximum(deg, 1.0)
        h = ((a0_ref[...] * inv[:, None]) @ wl_ref[0]
             + (a1_ref[...] * inv[:, None]) @ wl_ref[1]
             + x0_ref[...] @ wr_ref[0]
             + x1_ref[...] @ wr_ref[1]
             + b_ref[...])
        if last:
            o_ref[...] = h
        else:
            nrm = jnp.sqrt(jnp.sum(h * h, axis=1, keepdims=True))
            h = h / jnp.maximum(nrm, 1e-12)
            h = jnp.maximum(h, 0.0)
            o_ref[0] = h[:, :DH]
            o_ref[1] = h[:, DH:]

    if last:
        out_shape = jax.ShapeDtypeStruct((NP, DD), jnp.float32)
        out_spec = pl.BlockSpec((RBLK, DD), lambda i: (i, 0))
    else:
        out_shape = jax.ShapeDtypeStruct((2, NP, DH), jnp.float32)
        out_spec = pl.BlockSpec((2, RBLK, DH), lambda i: (0, i, 0))

    return pl.pallas_call(
        body,
        grid=(nblk,),
        in_specs=[
            pl.BlockSpec((NTILES, RBLK), lambda i: (0, i)),
            pl.BlockSpec((RBLK, DH), lambda i: (i, 0)),
            pl.BlockSpec((RBLK, DH), lambda i: (i + nblk, 0)),
            pl.BlockSpec((RBLK, DH), lambda i: (i, 0)),
            pl.BlockSpec((RBLK, DH), lambda i: (i + nblk, 0)),
            pl.BlockSpec((2, DH, DD), lambda i: (0, 0, 0)),
            pl.BlockSpec((2, DH, DD), lambda i: (0, 0, 0)),
            pl.BlockSpec((1, DD), lambda i: (0, 0)),
        ],
        out_specs=out_spec,
        out_shape=out_shape,
        compiler_params=pltpu.CompilerParams(
            dimension_semantics=("arbitrary",)),
    )(degp, aggf, aggf, xf, xf, wl2, wr2, bl2d)


def kernel(x, adjs, Wl0, bl0, Wr0, Wl1, bl1, Wr1, Wl2, bl2, Wr2):
    params = [(Wl0, bl0, Wr0), (Wl1, bl1, Wr1), (Wl2, bl2, Wr2)]
    xp = jnp.pad(x, ((0, NP - NN), (0, 0)))
    xf = xp.reshape(NP, 2, DH).transpose(1, 0, 2).reshape(2 * NP, DH)
    out = None
    for i in range(3):
        src = adjs[i, 0, 0]
        dst = adjs[i, 0, 1]
        Wl, bl, Wr = params[i]
        aggf, degp = _sc_aggregate(xf, src, dst)
        wl2 = Wl.reshape(2, DH, DD)
        wr2 = Wr.reshape(2, DH, DD)
        bl2d = bl.reshape(1, DD)
        last = i == 2
        y = _tc_update(aggf, degp, xf, wl2, bl2d, wr2, last)
        if last:
            out = y[:NN]
        else:
            xf = y.reshape(2 * NP, DH)
    return out
